# Initial kernel scaffold; baseline (speedup 1.0000x reference)
import jax
import jax.numpy as jnp
from jax.experimental import pallas as pl


def kernel(questions, question_length, images, img_relations, fact_features, fact_e1ids, fact_e2ids, fact_batch, glove, W_x, W_h, b_lstm, Wq_node, Wi_node, v_node, Wq_rel, Wr_rel, v_rel, W_msg1, W_self1, W_msg2, W_self2, W_img_att1, W_que_att1, v_att1, W_fact1, W_img_att2, W_que_att2, v_att2, W_fact2):
    raise NotImplementedError("write your pallas kernel here")


# speedup vs baseline: 5.5785x; 5.5785x over previous
import functools

import jax
import jax.numpy as jnp
from jax import lax
from jax.experimental import pallas as pl
from jax.experimental.pallas import tpu as pltpu
from jax.experimental.pallas import tpu_sc as plsc

F32 = jnp.float32
I32 = jnp.int32

N_FACT = 50000
E_FACT = 800000
E_PAD = 819200
E_SENTINEL = 1 << 20
FFW = 128
GW = 32
SW = 16
NP = 50176
NC = 2
NS = 16
ECH = 1024


def _mm(a, b):
    return lax.dot_general(a, b, (((1,), (0,)), ((), ())),
                           preferred_element_type=F32)


def _iota_eq(shape, f0, f1):
    r = f0(lax.broadcasted_iota(I32, shape, 0))
    c = f1(lax.broadcasted_iota(I32, shape, 1))
    return (r == c).astype(F32)


def _a_body(emb_ref, qlen_ref, img_ref, rel_ref,
            Wx_ref, Wh_ref, b_ref, Wqn_ref, Win_ref, vn_ref,
            Wqr_ref, Wrr_ref, vr_ref,
            Wm1t_ref, Wm1b_ref, Ws1_ref, Wm2t_ref, Wm2b_ref, Ws2_ref,
            Wia1_ref, Wqa1_ref, va1_ref, W1c_ref,
            Wia2_ref, Wqa2_ref, va2_ref, w2c_ref,
            ctx1p_ref, ctx2s_ref):
    Wx = Wx_ref[:]
    Wh = Wh_ref[:]
    bb = b_ref[:]
    qlen = qlen_ref[:]

    def step(t, carry):
        h, c, hl = carry
        x = emb_ref[pl.ds(t * 8, 8), :]
        g = _mm(x, Wx) + _mm(h, Wh) + bb
        ii = jax.nn.sigmoid(g[:, 0:512])
        ff = jax.nn.sigmoid(g[:, 512:1024])
        gg = jnp.tanh(g[:, 1024:1536])
        oo = jax.nn.sigmoid(g[:, 1536:2048])
        c = ff * c + ii * gg
        h = oo * jnp.tanh(c)
        hl = jnp.where(qlen - 1 == t, h, hl)
        return (h, c, hl)

    z = jnp.zeros((8, 512), F32)
    _, _, que = lax.fori_loop(0, 20, step, (z, z, z))

    ST = _iota_eq((288, 8), lambda r: r // 36, lambda c: c)
    S = _iota_eq((8, 288), lambda r: r, lambda c: c // 36)
    STQ = _iota_eq((10368, 8), lambda r: r // 1296, lambda c: c)

    imgs_raw = img_ref[:]
    nlog = jnp.tanh(_mm(imgs_raw, Win_ref[:]) + _mm(ST, _mm(que, Wqn_ref[:])))
    en = jnp.exp(_mm(nlog, vn_ref[:]))
    nav = en / _mm(ST, _mm(S, en))
    imgs = nav * imgs_raw

    rel = rel_ref[:]
    t2 = jnp.tanh(_mm(rel, Wrr_ref[:]) + _mm(STQ, _mm(que, Wqr_ref[:])))
    rav = _mm(t2, vr_ref[:])
    rels = rav * rel

    S36 = _iota_eq((36, 1296), lambda r: r, lambda c: c // 36)
    OH = _iota_eq((1296, 36), lambda r: r % 36, lambda c: c)
    ravo = rav * jnp.concatenate([OH] * 8, axis=0)
    relagg = jnp.concatenate(
        [_mm(S36, rels[b * 1296:(b + 1) * 1296, :]) for b in range(8)], axis=0)
    ATT = jnp.concatenate(
        [_mm(S36, ravo[b * 1296:(b + 1) * 1296, :]) for b in range(8)], axis=0)
    G36 = _iota_eq((36, 288), lambda r: r, lambda c: c % 36)
    BD = _iota_eq((288, 288), lambda r: r // 36, lambda c: c // 36)
    P = _mm(ATT, G36) * BD

    agg1 = _mm(_mm(P, imgs), Wm1t_ref[:]) + _mm(relagg, Wm1b_ref[:])
    h_img = jax.nn.relu(_mm(imgs, Ws1_ref[:]) + agg1)
    agg2 = _mm(_mm(P, h_img), Wm2t_ref[:]) + _mm(relagg, Wm2b_ref[:])
    h_img2 = jax.nn.relu(_mm(h_img, Ws2_ref[:]) + agg2)

    ea = jnp.tanh(_mm(h_img, Wia1_ref[:]) + _mm(ST, _mm(que, Wqa1_ref[:])))
    e1 = jnp.exp(_mm(ea, va1_ref[:]))
    al1 = e1 / _mm(ST, _mm(S, e1))
    ctx1 = _mm(S, al1 * h_img)
    ctx1p_ref[:] = _mm(ctx1, W1c_ref[:])

    eb = jnp.tanh(_mm(h_img2, Wia2_ref[:]) + _mm(ST, _mm(que, Wqa2_ref[:])))
    e2 = jnp.exp(_mm(eb, va2_ref[:]))
    al2 = e2 / _mm(ST, _mm(S, e2))
    ctx2 = _mm(S, al2 * h_img2)
    ctx2s_ref[:] = _mm(ctx2, w2c_ref[:])


def _run_a(emb2, qlen, images2, rel2, *weights):
    return pl.pallas_call(
        _a_body,
        out_shape=(jax.ShapeDtypeStruct((8, 300), F32),
                   jax.ShapeDtypeStruct((8, 1), F32)),
    )(emb2, qlen, images2, rel2, *weights)


def _make_seg_sum(ffw, nwin, split_by_core, name):
    ept = E_PAD // (NC * NS) if split_by_core else E_PAD // NS
    nch = ept // ECH
    rows_pt = NP // NS
    out_rows = 2 * NP if split_by_core else 2 * nwin * NP
    mesh = plsc.VectorSubcoreMesh(core_axis_name="c", subcore_axis_name="s",
                                  num_cores=NC, num_subcores=NS)

    @functools.partial(
        pl.kernel,
        out_type=jax.ShapeDtypeStruct((out_rows, ffw), F32),
        mesh=mesh,
        scratch_types=[
            pltpu.VMEM((ECH,), I32),
            pltpu.VMEM((ECH,), I32),
            pltpu.VMEM((128,), I32),
            pltpu.VMEM((128,), I32),
            pltpu.VMEM((128, ffw), F32),
            pltpu.VMEM((16, ffw), F32),
            pltpu.VMEM_SHARED((NP + 8, ffw), F32),
            pltpu.SemaphoreType.DMA,
        ],
        name=name,
        compiler_params=pltpu.CompilerParams(use_tc_tiling_on_sc=False),
    )
    def seg(e1_hbm, e2_hbm, tab_hbm, out_hbm,
            te1, te2, ce1, ce2, rows, zb, acc, sem):
        c = lax.axis_index("c")
        s = lax.axis_index("s")
        zerov = jnp.zeros((16,), F32)
        dumpv = jnp.full((16,), NP, I32)
        for r in range(16):
            for cc in range(ffw // 16):
                zb[r, pl.ds(cc * 16, 16)] = zerov

        for w in range(nwin):
            g = c * nwin + w
            goff = 0 if split_by_core else g * NP

            def zbody(z, _):
                off = pl.multiple_of(s * rows_pt + z * 16, 16)
                pltpu.sync_copy(zb, acc.at[pl.ds(off, 16)])
                return 0
            lax.fori_loop(0, rows_pt // 16, zbody, 0)
            plsc.subcore_barrier()

            def chunk(ch, _):
                if split_by_core:
                    base = (s * NC + c) * ept + ch * ECH
                else:
                    base = s * ept + ch * ECH
                base = pl.multiple_of(base, 8)
                pltpu.sync_copy(e1_hbm.at[pl.ds(base, ECH)], te1)
                pltpu.sync_copy(e2_hbm.at[pl.ds(base, ECH)], te2)
                for k in range(ECH // 128):
                    for v in range(8):
                        o = k * 128 + v * 16
                        ev1 = te1[pl.ds(o, 16)]
                        ev2 = te2[pl.ds(o, 16)]
                        ce1[pl.ds(v * 16, 16)] = ev1 + goff
                        ce2[pl.ds(v * 16, 16)] = jnp.where(ev2 < NP, ev2, dumpv)
                    pltpu.async_copy(tab_hbm.at[ce1], rows, sem).wait()
                    pltpu.sync_copy(rows, acc.at[ce2], add=True)
                return 0
            lax.fori_loop(0, nch, chunk, 0)

            plsc.subcore_barrier()
            src = pl.multiple_of(s * rows_pt, 8)
            if split_by_core:
                dst = c * NP + s * rows_pt
            else:
                dst = g * NP + s * rows_pt
            dst = pl.multiple_of(dst, 8)
            pltpu.sync_copy(acc.at[pl.ds(src, rows_pt)],
                            out_hbm.at[pl.ds(dst, rows_pt)])
            plsc.subcore_barrier()

    return seg


@functools.cache
def _seg_wide():
    return _make_seg_sum(GW, 2, False, "seg_sum_gcn1")


@functools.cache
def _seg_scalar():
    return _make_seg_sum(SW, 1, True, "seg_sum_gcn2")


def _seg_sum_sc_wide(e1p, e2p, table):
    return _seg_wide()(e1p, e2p, table)


def _seg_sum_sc_scalar(e1p, e2p, table):
    out = _seg_scalar()(e1p, e2p, table)
    return out[:NP, 0:1], out[NP:, 0:1]


def _c_body(ffp_ref, a0_ref, a1_ref, a2_ref, a3_ref, fb_ref, W1h_ref,
            W1a_ref, c1p_ref, wv2_ref, c2s_ref, out_ref):
    oh = (fb_ref[:] == lax.broadcasted_iota(I32, (512, 8), 1)).astype(F32)
    agg = jnp.concatenate([a0_ref[:], a1_ref[:], a2_ref[:], a3_ref[:]],
                          axis=1)
    h1 = jax.nn.relu(_mm(ffp_ref[:], W1h_ref[:]) + _mm(agg, W1a_ref[:])
                     + _mm(oh, c1p_ref[:]))
    hw = _mm(h1, wv2_ref[:])
    out_ref[:, 0:1] = hw[:, 0:1] + _mm(oh, c2s_ref[:])
    out_ref[:, 1:2] = hw[:, 1:2]


def _run_c(ffp, agg_stacked, fb2, W1hp, W1ap, ctx1p, wv2, ctx2s):
    full = lambda shape: pl.BlockSpec(shape, lambda i: (0, 0))
    row = lambda shape: pl.BlockSpec(shape, lambda i: (i, 0))
    gblk = lambda g: pl.BlockSpec((512, GW), lambda i, g=g: (g * (NP // 512) + i, 0))
    return pl.pallas_call(
        _c_body,
        grid=(NP // 512,),
        in_specs=[row((512, FFW)), gblk(0), gblk(1), gblk(2), gblk(3),
                  row((512, 1)),
                  full((FFW, 300)), full((FFW, 300)), full((8, 300)),
                  full((300, 2)), full((8, 1))],
        out_specs=row((512, 2)),
        out_shape=jax.ShapeDtypeStruct((NP, 2), F32),
    )(ffp, agg_stacked, agg_stacked, agg_stacked, agg_stacked, fb2,
      W1hp, W1ap, ctx1p, wv2, ctx2s)


def _e_body(t_ref, pa_ref, pb_ref, o_ref):
    x = jax.nn.sigmoid(t_ref[:] + pa_ref[:] + pb_ref[:])
    gid = (lax.broadcasted_iota(I32, (NP // 128, 128), 0) * 128
           + lax.broadcasted_iota(I32, (NP // 128, 128), 1))
    e = jnp.where(gid < N_FACT, jnp.exp(x), 0.0)
    o_ref[:] = e / jnp.sum(e)


def _run_e(t2, pa2, pb2):
    return pl.pallas_call(
        _e_body,
        out_shape=jax.ShapeDtypeStruct((NP // 128, 128), F32),
    )(t2, pa2, pb2)


def kernel(questions, question_length, images, img_relations, fact_features,
           fact_e1ids, fact_e2ids, fact_batch, glove, W_x, W_h, b_lstm,
           Wq_node, Wi_node, v_node, Wq_rel, Wr_rel, v_rel, W_msg1, W_self1,
           W_msg2, W_self2, W_img_att1, W_que_att1, v_att1, W_fact1,
           W_img_att2, W_que_att2, v_att2, W_fact2):
    emb = jnp.take(glove, questions, axis=0)
    emb2 = jnp.swapaxes(emb, 0, 1).reshape(160, 300)
    qlen = question_length.astype(I32).reshape(8, 1)
    images2 = images.reshape(288, 2048)
    rel2 = jnp.swapaxes(img_relations, 1, 2).reshape(10368, 7)

    weights = (
        W_x, W_h, b_lstm.reshape(1, 2048), Wq_node, Wi_node, v_node,
        Wq_rel, Wr_rel, v_rel,
        W_msg1[:2048], W_msg1[2048:], W_self1,
        W_msg2[:512], W_msg2[512:], W_self2,
        W_img_att1, W_que_att1, v_att1, W_fact1[200:],
        W_img_att2, W_que_att2, v_att2, W_fact2[600:],
    )
    ctx1p, ctx2s = _run_a(emb2, qlen, images2, rel2, *weights)

    e1p = jnp.concatenate([fact_e1ids.astype(I32),
                           jnp.zeros((E_PAD - E_FACT,), I32)])
    e2p = jnp.concatenate([fact_e2ids.astype(I32),
                           jnp.full((E_PAD - E_FACT,), E_SENTINEL, I32)])
    ffp = jnp.pad(fact_features, ((0, NP - N_FACT), (0, FFW - 100)))
    tab = jnp.concatenate([ffp[:, g * GW:(g + 1) * GW] for g in range(4)])

    agg1 = _seg_sum_sc_wide(e1p, e2p, tab)

    fb2 = jnp.pad(fact_batch.astype(I32), (0, NP - N_FACT)).reshape(NP, 1)
    W1hp = jnp.pad(W_fact1[:100], ((0, FFW - 100), (0, 0)))
    W1ap = jnp.pad(W_fact1[100:200], ((0, FFW - 100), (0, 0)))
    wv2 = jnp.concatenate([W_fact2[:300], W_fact2[300:600]], axis=1)
    ts = _run_c(ffp, agg1, fb2, W1hp, W1ap, ctx1p, wv2, ctx2s)

    sp = jnp.pad(ts[:, 1:2], ((0, 0), (0, SW - 1)))
    pa, pb = _seg_sum_sc_scalar(e1p, e2p, sp)

    out = _run_e(ts[:, 0:1].reshape(NP // 128, 128),
                 pa.reshape(NP // 128, 128), pb.reshape(NP // 128, 128))
    return out.reshape(NP, 1)[:N_FACT]


# speedup vs baseline: 7.3632x; 1.3199x over previous
import functools

import jax
import jax.numpy as jnp
from jax import lax
from jax.experimental import pallas as pl
from jax.experimental.pallas import tpu as pltpu
from jax.experimental.pallas import tpu_sc as plsc

F32 = jnp.float32
I32 = jnp.int32

N_FACT = 50000
E_FACT = 800000
E_PAD = 819200
E_SENTINEL = 1 << 20
FFW = 128
GW = 32
SW = 16
NP = 50176
NC = 2
NS = 16
ECH = 1280
NBUF = 4


def _mm(a, b):
    return lax.dot_general(a, b, (((1,), (0,)), ((), ())),
                           preferred_element_type=F32)


def _iota_eq(shape, f0, f1):
    r = f0(lax.broadcasted_iota(I32, shape, 0))
    c = f1(lax.broadcasted_iota(I32, shape, 1))
    return (r == c).astype(F32)


def _a_body(emb_ref, qlen_ref, img_ref, rel_ref,
            Wx_ref, Wh_ref, b_ref, Wqn_ref, Win_ref, vn_ref,
            Wqr_ref, Wrr_ref, vr_ref,
            Wm1t_ref, Wm1b_ref, Ws1_ref, Wm2t_ref, Wm2b_ref, Ws2_ref,
            Wia1_ref, Wqa1_ref, va1_ref, W1c_ref,
            Wia2_ref, Wqa2_ref, va2_ref, w2c_ref,
            ctx1p_ref, ctx2s_ref):
    Wx = Wx_ref[:]
    Wh = Wh_ref[:]
    bb = b_ref[:]
    qlen = qlen_ref[:]

    def step(t, carry):
        h, c, hl = carry
        x = emb_ref[pl.ds(t * 8, 8), :]
        g = _mm(x, Wx) + _mm(h, Wh) + bb
        ii = jax.nn.sigmoid(g[:, 0:512])
        ff = jax.nn.sigmoid(g[:, 512:1024])
        gg = jnp.tanh(g[:, 1024:1536])
        oo = jax.nn.sigmoid(g[:, 1536:2048])
        c = ff * c + ii * gg
        h = oo * jnp.tanh(c)
        hl = jnp.where(qlen - 1 == t, h, hl)
        return (h, c, hl)

    z = jnp.zeros((8, 512), F32)
    _, _, que = lax.fori_loop(0, 20, step, (z, z, z))

    ST = _iota_eq((288, 8), lambda r: r // 36, lambda c: c)
    S = _iota_eq((8, 288), lambda r: r, lambda c: c // 36)
    STQ = _iota_eq((10368, 8), lambda r: r // 1296, lambda c: c)

    imgs_raw = img_ref[:]
    nlog = jnp.tanh(_mm(imgs_raw, Win_ref[:]) + _mm(ST, _mm(que, Wqn_ref[:])))
    en = jnp.exp(_mm(nlog, vn_ref[:]))
    nav = en / _mm(ST, _mm(S, en))
    imgs = nav * imgs_raw

    rel = rel_ref[:]
    t2 = jnp.tanh(_mm(rel, Wrr_ref[:]) + _mm(STQ, _mm(que, Wqr_ref[:])))
    rav = _mm(t2, vr_ref[:])
    rels = rav * rel

    S36 = _iota_eq((36, 1296), lambda r: r, lambda c: c // 36)
    OH = _iota_eq((1296, 36), lambda r: r % 36, lambda c: c)
    ravo = rav * jnp.concatenate([OH] * 8, axis=0)
    relagg = jnp.concatenate(
        [_mm(S36, rels[b * 1296:(b + 1) * 1296, :]) for b in range(8)], axis=0)
    ATT = jnp.concatenate(
        [_mm(S36, ravo[b * 1296:(b + 1) * 1296, :]) for b in range(8)], axis=0)
    G36 = _iota_eq((36, 288), lambda r: r, lambda c: c % 36)
    BD = _iota_eq((288, 288), lambda r: r // 36, lambda c: c // 36)
    P = _mm(ATT, G36) * BD

    agg1 = _mm(_mm(P, imgs), Wm1t_ref[:]) + _mm(relagg, Wm1b_ref[:])
    h_img = jax.nn.relu(_mm(imgs, Ws1_ref[:]) + agg1)
    agg2 = _mm(_mm(P, h_img), Wm2t_ref[:]) + _mm(relagg, Wm2b_ref[:])
    h_img2 = jax.nn.relu(_mm(h_img, Ws2_ref[:]) + agg2)

    ea = jnp.tanh(_mm(h_img, Wia1_ref[:]) + _mm(ST, _mm(que, Wqa1_ref[:])))
    e1 = jnp.exp(_mm(ea, va1_ref[:]))
    al1 = e1 / _mm(ST, _mm(S, e1))
    ctx1 = _mm(S, al1 * h_img)
    ctx1p_ref[:] = _mm(ctx1, W1c_ref[:])

    eb = jnp.tanh(_mm(h_img2, Wia2_ref[:]) + _mm(ST, _mm(que, Wqa2_ref[:])))
    e2 = jnp.exp(_mm(eb, va2_ref[:]))
    al2 = e2 / _mm(ST, _mm(S, e2))
    ctx2 = _mm(S, al2 * h_img2)
    ctx2s_ref[:] = _mm(ctx2, w2c_ref[:])


def _run_a(emb2, qlen, images2, rel2, *weights):
    return pl.pallas_call(
        _a_body,
        out_shape=(jax.ShapeDtypeStruct((8, 300), F32),
                   jax.ShapeDtypeStruct((8, 1), F32)),
    )(emb2, qlen, images2, rel2, *weights)


def _make_seg_sum(ffw, nwin, split_by_core, name):
    ept = E_PAD // (NC * NS) if split_by_core else E_PAD // NS
    nch = ept // ECH
    rows_pt = NP // NS
    out_rows = 2 * NP if split_by_core else 2 * nwin * NP
    mesh = plsc.VectorSubcoreMesh(core_axis_name="c", subcore_axis_name="s",
                                  num_cores=NC, num_subcores=NS)

    G = ECH // 128

    @functools.partial(
        pl.kernel,
        out_type=jax.ShapeDtypeStruct((out_rows, ffw), F32),
        mesh=mesh,
        scratch_types=(
            [pltpu.VMEM((ECH,), I32),
             pltpu.VMEM((ECH,), I32)]
            + [pltpu.VMEM((128,), I32) for _ in range(NBUF)]
            + [pltpu.VMEM((128,), I32) for _ in range(NBUF)]
            + [pltpu.VMEM((128, ffw), F32) for _ in range(NBUF)]
            + [pltpu.VMEM((16, ffw), F32)]
            + [pltpu.VMEM_SHARED((NP + 8, ffw), F32)]
            + [pltpu.SemaphoreType.DMA for _ in range(2 * NBUF)]
        ),
        name=name,
        compiler_params=pltpu.CompilerParams(use_tc_tiling_on_sc=False),
    )
    def seg(e1_hbm, e2_hbm, tab_hbm, out_hbm, te1, te2, *sc):
        ce1s = sc[0:NBUF]
        ce2s = sc[NBUF:2 * NBUF]
        rowss = sc[2 * NBUF:3 * NBUF]
        zb = sc[3 * NBUF]
        acc = sc[3 * NBUF + 1]
        gsems = sc[3 * NBUF + 2:3 * NBUF + 2 + NBUF]
        ssems = sc[3 * NBUF + 2 + NBUF:3 * NBUF + 2 + 2 * NBUF]
        c = lax.axis_index("c")
        s = lax.axis_index("s")
        zerov = jnp.zeros((16,), F32)
        dumpv = jnp.full((16,), NP, I32)
        for r in range(16):
            for cc in range(ffw // 16):
                zb[r, pl.ds(cc * 16, 16)] = zerov

        for w in range(nwin):
            g = c * nwin + w
            goff = 0 if split_by_core else g * NP

            def zbody(z, _):
                off = pl.multiple_of(s * rows_pt + z * 16, 16)
                pltpu.sync_copy(zb, acc.at[pl.ds(off, 16)])
                return 0
            lax.fori_loop(0, rows_pt // 16, zbody, 0)
            plsc.subcore_barrier()

            def chunk(ch, _):
                if split_by_core:
                    base = (s * NC + c) * ept + ch * ECH
                else:
                    base = s * ept + ch * ECH
                base = pl.multiple_of(base, 8)
                pltpu.sync_copy(e1_hbm.at[pl.ds(base, ECH)], te1)
                pltpu.sync_copy(e2_hbm.at[pl.ds(base, ECH)], te2)

                def scat(k):
                    b = k % NBUF
                    return pltpu.async_copy(rowss[b], acc.at[ce2s[b]],
                                            ssems[b], add=True)

                gds, sds = {}, {}
                for k in range(G):
                    b = k % NBUF
                    if k >= NBUF:
                        sds[k - NBUF].wait()
                    for v in range(8):
                        o = k * 128 + v * 16
                        ev1 = te1[pl.ds(o, 16)]
                        ev2 = te2[pl.ds(o, 16)]
                        ce1s[b][pl.ds(v * 16, 16)] = ev1 + goff
                        ce2s[b][pl.ds(v * 16, 16)] = jnp.where(ev2 < NP,
                                                               ev2, dumpv)
                    gds[k] = pltpu.async_copy(tab_hbm.at[ce1s[b]], rowss[b],
                                              gsems[b])
                    kp = k - (NBUF - 1)
                    if kp >= 0:
                        gds[kp].wait()
                        sds[kp] = scat(kp)
                for kp in range(max(0, G - (NBUF - 1)), G):
                    gds[kp].wait()
                    sds[kp] = scat(kp)
                for kp in range(max(0, G - NBUF), G):
                    sds[kp].wait()
                return 0
            lax.fori_loop(0, nch, chunk, 0)

            plsc.subcore_barrier()
            src = pl.multiple_of(s * rows_pt, 8)
            if split_by_core:
                dst = c * NP + s * rows_pt
            else:
                dst = g * NP + s * rows_pt
            dst = pl.multiple_of(dst, 8)
            pltpu.sync_copy(acc.at[pl.ds(src, rows_pt)],
                            out_hbm.at[pl.ds(dst, rows_pt)])
            plsc.subcore_barrier()

    return seg


@functools.cache
def _seg_wide():
    return _make_seg_sum(GW, 2, False, "seg_sum_gcn1")


@functools.cache
def _seg_scalar():
    return _make_seg_sum(SW, 1, True, "seg_sum_gcn2")


def _seg_sum_sc_wide(e1p, e2p, table):
    return _seg_wide()(e1p, e2p, table)


def _seg_sum_sc_scalar(e1p, e2p, table):
    out = _seg_scalar()(e1p, e2p, table)
    return out[:NP, 0:1], out[NP:, 0:1]


def _c_body(ffp_ref, a0_ref, a1_ref, a2_ref, a3_ref, fb_ref, W1h_ref,
            W1a_ref, c1p_ref, wv2_ref, c2s_ref, out_ref):
    oh = (fb_ref[:] == lax.broadcasted_iota(I32, (512, 8), 1)).astype(F32)
    agg = jnp.concatenate([a0_ref[:], a1_ref[:], a2_ref[:], a3_ref[:]],
                          axis=1)
    h1 = jax.nn.relu(_mm(ffp_ref[:], W1h_ref[:]) + _mm(agg, W1a_ref[:])
                     + _mm(oh, c1p_ref[:]))
    hw = _mm(h1, wv2_ref[:])
    out_ref[:, 0:1] = hw[:, 0:1] + _mm(oh, c2s_ref[:])
    out_ref[:, 1:2] = hw[:, 1:2]


def _run_c(ffp, agg_stacked, fb2, W1hp, W1ap, ctx1p, wv2, ctx2s):
    full = lambda shape: pl.BlockSpec(shape, lambda i: (0, 0))
    row = lambda shape: pl.BlockSpec(shape, lambda i: (i, 0))
    gblk = lambda g: pl.BlockSpec((512, GW), lambda i, g=g: (g * (NP // 512) + i, 0))
    return pl.pallas_call(
        _c_body,
        grid=(NP // 512,),
        in_specs=[row((512, FFW)), gblk(0), gblk(1), gblk(2), gblk(3),
                  row((512, 1)),
                  full((FFW, 300)), full((FFW, 300)), full((8, 300)),
                  full((300, 2)), full((8, 1))],
        out_specs=row((512, 2)),
        out_shape=jax.ShapeDtypeStruct((NP, 2), F32),
    )(ffp, agg_stacked, agg_stacked, agg_stacked, agg_stacked, fb2,
      W1hp, W1ap, ctx1p, wv2, ctx2s)


def _e_body(t_ref, pa_ref, pb_ref, o_ref):
    x = jax.nn.sigmoid(t_ref[:] + pa_ref[:] + pb_ref[:])
    gid = (lax.broadcasted_iota(I32, (NP // 128, 128), 0) * 128
           + lax.broadcasted_iota(I32, (NP // 128, 128), 1))
    e = jnp.where(gid < N_FACT, jnp.exp(x), 0.0)
    o_ref[:] = e / jnp.sum(e)


def _run_e(t2, pa2, pb2):
    return pl.pallas_call(
        _e_body,
        out_shape=jax.ShapeDtypeStruct((NP // 128, 128), F32),
    )(t2, pa2, pb2)


def kernel(questions, question_length, images, img_relations, fact_features,
           fact_e1ids, fact_e2ids, fact_batch, glove, W_x, W_h, b_lstm,
           Wq_node, Wi_node, v_node, Wq_rel, Wr_rel, v_rel, W_msg1, W_self1,
           W_msg2, W_self2, W_img_att1, W_que_att1, v_att1, W_fact1,
           W_img_att2, W_que_att2, v_att2, W_fact2):
    emb = jnp.take(glove, questions, axis=0)
    emb2 = jnp.swapaxes(emb, 0, 1).reshape(160, 300)
    qlen = question_length.astype(I32).reshape(8, 1)
    images2 = images.reshape(288, 2048)
    rel2 = jnp.swapaxes(img_relations, 1, 2).reshape(10368, 7)

    weights = (
        W_x, W_h, b_lstm.reshape(1, 2048), Wq_node, Wi_node, v_node,
        Wq_rel, Wr_rel, v_rel,
        W_msg1[:2048], W_msg1[2048:], W_self1,
        W_msg2[:512], W_msg2[512:], W_self2,
        W_img_att1, W_que_att1, v_att1, W_fact1[200:],
        W_img_att2, W_que_att2, v_att2, W_fact2[600:],
    )
    ctx1p, ctx2s = _run_a(emb2, qlen, images2, rel2, *weights)

    e1p = jnp.concatenate([fact_e1ids.astype(I32),
                           jnp.zeros((E_PAD - E_FACT,), I32)])
    e2p = jnp.concatenate([fact_e2ids.astype(I32),
                           jnp.full((E_PAD - E_FACT,), E_SENTINEL, I32)])
    ffp = jnp.pad(fact_features, ((0, NP - N_FACT), (0, FFW - 100)))
    tab = jnp.concatenate([ffp[:, g * GW:(g + 1) * GW] for g in range(4)])

    agg1 = _seg_sum_sc_wide(e1p, e2p, tab)

    fb2 = jnp.pad(fact_batch.astype(I32), (0, NP - N_FACT)).reshape(NP, 1)
    W1hp = jnp.pad(W_fact1[:100], ((0, FFW - 100), (0, 0)))
    W1ap = jnp.pad(W_fact1[100:200], ((0, FFW - 100), (0, 0)))
    wv2 = jnp.concatenate([W_fact2[:300], W_fact2[300:600]], axis=1)
    ts = _run_c(ffp, agg1, fb2, W1hp, W1ap, ctx1p, wv2, ctx2s)

    sp = jnp.pad(ts[:, 1:2], ((0, 0), (0, SW - 1)))
    pa, pb = _seg_sum_sc_scalar(e1p, e2p, sp)

    out = _run_e(ts[:, 0:1].reshape(NP // 128, 128),
                 pa.reshape(NP // 128, 128), pb.reshape(NP // 128, 128))
    return out.reshape(NP, 1)[:N_FACT]


# speedup vs baseline: 8.1711x; 1.1097x over previous
import functools

import jax
import jax.numpy as jnp
from jax import lax
from jax.experimental import pallas as pl
from jax.experimental.pallas import tpu as pltpu
from jax.experimental.pallas import tpu_sc as plsc

F32 = jnp.float32
I32 = jnp.int32

N_FACT = 50000
E_FACT = 800000
E_PAD = 819200
E_SENTINEL = 1 << 20
FFW = 128
GW = 32
SW = 16
NP = 50176
NC = 2
NS = 16
ECH = 1280
NBUF = 4


def _mm(a, b):
    return lax.dot_general(a, b, (((1,), (0,)), ((), ())),
                           preferred_element_type=F32)


def _iota_eq(shape, f0, f1):
    r = f0(lax.broadcasted_iota(I32, shape, 0))
    c = f1(lax.broadcasted_iota(I32, shape, 1))
    return (r == c).astype(F32)


def _a_body(emb_ref, qlen_ref, img_ref, rel_ref,
            Wx_ref, Wh_ref, b_ref, Wqn_ref, Win_ref, vn_ref,
            Wqr_ref, Wrr_ref, vr_ref,
            Wm1t_ref, Wm1b_ref, Ws1_ref, Wm2t_ref, Wm2b_ref, Ws2_ref,
            Wia1_ref, Wqa1_ref, va1_ref, W1c_ref,
            Wia2_ref, Wqa2_ref, va2_ref, w2c_ref,
            ctx1p_ref, ctx2s_ref):
    Wx = Wx_ref[:]
    Wh = Wh_ref[:]
    bb = b_ref[:]
    qlen = qlen_ref[:]

    def step(t, carry):
        h, c, hl = carry
        x = emb_ref[pl.ds(t * 8, 8), :]
        g = _mm(x, Wx) + _mm(h, Wh) + bb
        ii = jax.nn.sigmoid(g[:, 0:512])
        ff = jax.nn.sigmoid(g[:, 512:1024])
        gg = jnp.tanh(g[:, 1024:1536])
        oo = jax.nn.sigmoid(g[:, 1536:2048])
        c = ff * c + ii * gg
        h = oo * jnp.tanh(c)
        hl = jnp.where(qlen - 1 == t, h, hl)
        return (h, c, hl)

    z = jnp.zeros((8, 512), F32)
    _, _, que = lax.fori_loop(0, 20, step, (z, z, z))

    ST = _iota_eq((288, 8), lambda r: r // 36, lambda c: c)
    S = _iota_eq((8, 288), lambda r: r, lambda c: c // 36)
    STQ = _iota_eq((10368, 8), lambda r: r // 1296, lambda c: c)

    imgs_raw = img_ref[:]
    nlog = jnp.tanh(_mm(imgs_raw, Win_ref[:]) + _mm(ST, _mm(que, Wqn_ref[:])))
    en = jnp.exp(_mm(nlog, vn_ref[:]))
    nav = en / _mm(ST, _mm(S, en))
    imgs = nav * imgs_raw

    rel = rel_ref[:]
    t2 = jnp.tanh(_mm(rel, Wrr_ref[:]) + _mm(STQ, _mm(que, Wqr_ref[:])))
    rav = _mm(t2, vr_ref[:])
    rels = rav * rel

    S36 = _iota_eq((36, 1296), lambda r: r, lambda c: c // 36)
    OH = _iota_eq((1296, 36), lambda r: r % 36, lambda c: c)
    ravo = rav * jnp.concatenate([OH] * 8, axis=0)
    relagg = jnp.concatenate(
        [_mm(S36, rels[b * 1296:(b + 1) * 1296, :]) for b in range(8)], axis=0)
    ATT = jnp.concatenate(
        [_mm(S36, ravo[b * 1296:(b + 1) * 1296, :]) for b in range(8)], axis=0)
    G36 = _iota_eq((36, 288), lambda r: r, lambda c: c % 36)
    BD = _iota_eq((288, 288), lambda r: r // 36, lambda c: c // 36)
    P = _mm(ATT, G36) * BD

    agg1 = _mm(_mm(P, imgs), Wm1t_ref[:]) + _mm(relagg, Wm1b_ref[:])
    h_img = jax.nn.relu(_mm(imgs, Ws1_ref[:]) + agg1)
    agg2 = _mm(_mm(P, h_img), Wm2t_ref[:]) + _mm(relagg, Wm2b_ref[:])
    h_img2 = jax.nn.relu(_mm(h_img, Ws2_ref[:]) + agg2)

    ea = jnp.tanh(_mm(h_img, Wia1_ref[:]) + _mm(ST, _mm(que, Wqa1_ref[:])))
    e1 = jnp.exp(_mm(ea, va1_ref[:]))
    al1 = e1 / _mm(ST, _mm(S, e1))
    ctx1 = _mm(S, al1 * h_img)
    ctx1p_ref[:] = _mm(ctx1, W1c_ref[:])

    eb = jnp.tanh(_mm(h_img2, Wia2_ref[:]) + _mm(ST, _mm(que, Wqa2_ref[:])))
    e2 = jnp.exp(_mm(eb, va2_ref[:]))
    al2 = e2 / _mm(ST, _mm(S, e2))
    ctx2 = _mm(S, al2 * h_img2)
    ctx2s_ref[:] = _mm(ctx2, w2c_ref[:])


def _run_a(emb2, qlen, images2, rel2, *weights):
    return pl.pallas_call(
        _a_body,
        out_shape=(jax.ShapeDtypeStruct((8, 300), F32),
                   jax.ShapeDtypeStruct((8, 1), F32)),
    )(emb2, qlen, images2, rel2, *weights)


def _make_seg_sum(ffw, nwin, split_by_core, name):
    ept = E_PAD // (NC * NS) if split_by_core else E_PAD // NS
    nch = ept // ECH
    rows_pt = NP // NS
    out_rows = 2 * NP if split_by_core else 2 * nwin * NP
    mesh = plsc.VectorSubcoreMesh(core_axis_name="c", subcore_axis_name="s",
                                  num_cores=NC, num_subcores=NS)

    G = ECH // 128

    @functools.partial(
        pl.kernel,
        out_type=jax.ShapeDtypeStruct((out_rows, ffw), F32),
        mesh=mesh,
        scratch_types=(
            [pltpu.VMEM((ECH,), I32),
             pltpu.VMEM((ECH,), I32)]
            + [pltpu.VMEM((128,), I32) for _ in range(NBUF)]
            + [pltpu.VMEM((128,), I32) for _ in range(NBUF)]
            + [pltpu.VMEM((128, ffw), F32) for _ in range(NBUF)]
            + [pltpu.VMEM((16, ffw), F32)]
            + [pltpu.VMEM_SHARED((NP + 8, ffw), F32)]
            + [pltpu.SemaphoreType.DMA for _ in range(2 * NBUF)]
        ),
        name=name,
        compiler_params=pltpu.CompilerParams(use_tc_tiling_on_sc=False),
    )
    def seg(e1_hbm, e2_hbm, tab_hbm, out_hbm, te1, te2, *sc):
        ce1s = sc[0:NBUF]
        ce2s = sc[NBUF:2 * NBUF]
        rowss = sc[2 * NBUF:3 * NBUF]
        zb = sc[3 * NBUF]
        acc = sc[3 * NBUF + 1]
        gsems = sc[3 * NBUF + 2:3 * NBUF + 2 + NBUF]
        ssems = sc[3 * NBUF + 2 + NBUF:3 * NBUF + 2 + 2 * NBUF]
        c = lax.axis_index("c")
        s = lax.axis_index("s")
        zerov = jnp.zeros((16,), F32)
        dumpv = jnp.full((16,), NP, I32)
        for r in range(16):
            for cc in range(ffw // 16):
                zb[r, pl.ds(cc * 16, 16)] = zerov

        for w in range(nwin):
            g = c * nwin + w
            goff = 0 if split_by_core else g * NP

            def zbody(z, _):
                off = pl.multiple_of(s * rows_pt + z * 16, 16)
                pltpu.sync_copy(zb, acc.at[pl.ds(off, 16)])
                return 0
            lax.fori_loop(0, rows_pt // 16, zbody, 0)
            plsc.subcore_barrier()

            def chunk(ch, _):
                if split_by_core:
                    base = (s * NC + c) * ept + ch * ECH
                else:
                    base = s * ept + ch * ECH
                base = pl.multiple_of(base, 8)
                pltpu.sync_copy(e1_hbm.at[pl.ds(base, ECH)], te1)
                pltpu.sync_copy(e2_hbm.at[pl.ds(base, ECH)], te2)

                def scat(k):
                    b = k % NBUF
                    return pltpu.async_copy(rowss[b], acc.at[ce2s[b]],
                                            ssems[b], add=True)

                gds, sds = {}, {}
                for k in range(G):
                    b = k % NBUF
                    if k >= NBUF:
                        sds[k - NBUF].wait()
                    for v in range(8):
                        o = k * 128 + v * 16
                        ev1 = te1[pl.ds(o, 16)]
                        ev2 = te2[pl.ds(o, 16)]
                        ce1s[b][pl.ds(v * 16, 16)] = ev1 + goff
                        ce2s[b][pl.ds(v * 16, 16)] = jnp.where(ev2 < NP,
                                                               ev2, dumpv)
                    gds[k] = pltpu.async_copy(tab_hbm.at[ce1s[b]], rowss[b],
                                              gsems[b])
                    kp = k - (NBUF - 1)
                    if kp >= 0:
                        gds[kp].wait()
                        sds[kp] = scat(kp)
                for kp in range(max(0, G - (NBUF - 1)), G):
                    gds[kp].wait()
                    sds[kp] = scat(kp)
                for kp in range(max(0, G - NBUF), G):
                    sds[kp].wait()
                return 0
            lax.fori_loop(0, nch, chunk, 0)

            plsc.subcore_barrier()
            src = pl.multiple_of(s * rows_pt, 8)
            if split_by_core:
                dst = c * NP + s * rows_pt
            else:
                dst = g * NP + s * rows_pt
            dst = pl.multiple_of(dst, 8)
            pltpu.sync_copy(acc.at[pl.ds(src, rows_pt)],
                            out_hbm.at[pl.ds(dst, rows_pt)])
            plsc.subcore_barrier()

    return seg


@functools.cache
def _seg_wide():
    return _make_seg_sum(GW, 2, False, "seg_sum_gcn1")


@functools.cache
def _seg_scalar():
    return _make_seg_sum(SW, 1, True, "seg_sum_gcn2")


def _seg_sum_sc_wide(e1p, e2p, table):
    return _seg_wide()(e1p, e2p, table)


def _seg_sum_sc_scalar(e1p, e2p, table):
    out = _seg_scalar()(e1p, e2p, table)
    return out[:NP, 0:1], out[NP:, 0:1]


def _c_body(f0_ref, f1_ref, f2_ref, f3_ref, a0_ref, a1_ref, a2_ref, a3_ref,
            fb_ref, W1h_ref, W1a_ref, c1p_ref, wv2_ref, c2s_ref, out_ref):
    oh = (fb_ref[:] == lax.broadcasted_iota(I32, (512, 8), 1)).astype(F32)
    ff = jnp.concatenate([f0_ref[:], f1_ref[:], f2_ref[:], f3_ref[:]],
                         axis=1)
    agg = jnp.concatenate([a0_ref[:], a1_ref[:], a2_ref[:], a3_ref[:]],
                          axis=1)
    h1 = jax.nn.relu(_mm(ff, W1h_ref[:]) + _mm(agg, W1a_ref[:])
                     + _mm(oh, c1p_ref[:]))
    hw = _mm(h1, wv2_ref[:])
    out_ref[:, 0:1] = hw[:, 0:1] + _mm(oh, c2s_ref[:])
    out_ref[:, 1:2] = hw[:, 1:2]


def _run_c(tab, agg_stacked, fb2, W1hp, W1ap, ctx1p, wv2, ctx2s):
    full = lambda shape: pl.BlockSpec(shape, lambda i: (0, 0))
    row = lambda shape: pl.BlockSpec(shape, lambda i: (i, 0))
    gblk = lambda g: pl.BlockSpec((512, GW), lambda i, g=g: (g * (NP // 512) + i, 0))
    return pl.pallas_call(
        _c_body,
        grid=(NP // 512,),
        in_specs=[gblk(0), gblk(1), gblk(2), gblk(3),
                  gblk(0), gblk(1), gblk(2), gblk(3),
                  row((512, 1)),
                  full((FFW, 300)), full((FFW, 300)), full((8, 300)),
                  full((300, 2)), full((8, 1))],
        out_specs=row((512, 2)),
        out_shape=jax.ShapeDtypeStruct((NP, 2), F32),
    )(tab, tab, tab, tab, agg_stacked, agg_stacked, agg_stacked, agg_stacked,
      fb2, W1hp, W1ap, ctx1p, wv2, ctx2s)


def _e_body(t_ref, pa_ref, pb_ref, o_ref):
    x = jax.nn.sigmoid(t_ref[:] + pa_ref[:] + pb_ref[:])
    gid = (lax.broadcasted_iota(I32, (NP // 128, 128), 0) * 128
           + lax.broadcasted_iota(I32, (NP // 128, 128), 1))
    e = jnp.where(gid < N_FACT, jnp.exp(x), 0.0)
    o_ref[:] = e / jnp.sum(e)


def _run_e(t2, pa2, pb2):
    return pl.pallas_call(
        _e_body,
        out_shape=jax.ShapeDtypeStruct((NP // 128, 128), F32),
    )(t2, pa2, pb2)


def kernel(questions, question_length, images, img_relations, fact_features,
           fact_e1ids, fact_e2ids, fact_batch, glove, W_x, W_h, b_lstm,
           Wq_node, Wi_node, v_node, Wq_rel, Wr_rel, v_rel, W_msg1, W_self1,
           W_msg2, W_self2, W_img_att1, W_que_att1, v_att1, W_fact1,
           W_img_att2, W_que_att2, v_att2, W_fact2):
    emb = jnp.take(glove, questions, axis=0)
    emb2 = jnp.swapaxes(emb, 0, 1).reshape(160, 300)
    qlen = question_length.astype(I32).reshape(8, 1)
    images2 = images.reshape(288, 2048)
    rel2 = jnp.swapaxes(img_relations, 1, 2).reshape(10368, 7)

    weights = (
        W_x, W_h, b_lstm.reshape(1, 2048), Wq_node, Wi_node, v_node,
        Wq_rel, Wr_rel, v_rel,
        W_msg1[:2048], W_msg1[2048:], W_self1,
        W_msg2[:512], W_msg2[512:], W_self2,
        W_img_att1, W_que_att1, v_att1, W_fact1[200:],
        W_img_att2, W_que_att2, v_att2, W_fact2[600:],
    )
    ctx1p, ctx2s = _run_a(emb2, qlen, images2, rel2, *weights)

    e1p = jnp.concatenate([fact_e1ids.astype(I32),
                           jnp.zeros((E_PAD - E_FACT,), I32)])
    e2p = jnp.concatenate([fact_e2ids.astype(I32),
                           jnp.full((E_PAD - E_FACT,), E_SENTINEL, I32)])
    tab = jnp.concatenate(
        [jnp.pad(fact_features[:, g * GW:min((g + 1) * GW, 100)],
                 ((0, NP - N_FACT), (0, max(0, (g + 1) * GW - 100))))
         for g in range(4)])

    agg1 = _seg_sum_sc_wide(e1p, e2p, tab)

    fb2 = jnp.pad(fact_batch.astype(I32), (0, NP - N_FACT)).reshape(NP, 1)
    W1hp = jnp.pad(W_fact1[:100], ((0, FFW - 100), (0, 0)))
    W1ap = jnp.pad(W_fact1[100:200], ((0, FFW - 100), (0, 0)))
    wv2 = jnp.concatenate([W_fact2[:300], W_fact2[300:600]], axis=1)
    ts = _run_c(tab, agg1, fb2, W1hp, W1ap, ctx1p, wv2, ctx2s)

    sp = jnp.pad(ts[:, 1:2], ((0, 0), (0, SW - 1)))
    pa, pb = _seg_sum_sc_scalar(e1p, e2p, sp)

    out = _run_e(ts[:, 0:1].reshape(NP // 128, 128),
                 pa.reshape(NP // 128, 128), pb.reshape(NP // 128, 128))
    return out.reshape(NP, 1)[:N_FACT]


# speedup vs baseline: 15.6782x; 1.9187x over previous
import functools

import jax
import jax.numpy as jnp
from jax import lax
from jax.experimental import pallas as pl
from jax.experimental.pallas import tpu as pltpu
from jax.experimental.pallas import tpu_sc as plsc

F32 = jnp.float32
I32 = jnp.int32

N_FACT = 50000
E_FACT = 800000
E_PAD = 819200
E_SENTINEL = 1 << 20
FFW = 128
GW = 32
GWB = 64
SW = 16
NP = 50176
NC = 2
NS = 16
ECH = 1280
NBUF = 4


def _mm(a, b):
    return lax.dot_general(a, b, (((1,), (0,)), ((), ())),
                           preferred_element_type=F32)


def _iota_eq(shape, f0, f1):
    r = f0(lax.broadcasted_iota(I32, shape, 0))
    c = f1(lax.broadcasted_iota(I32, shape, 1))
    return (r == c).astype(F32)


def _a_body(emb_ref, qlen_ref, img_ref, rel_ref,
            Wx_ref, Wh_ref, b_ref, Wqn_ref, Win_ref, vn_ref,
            Wqr_ref, Wrr_ref, vr_ref,
            Wm1t_ref, Wm1b_ref, Ws1_ref, Wm2t_ref, Wm2b_ref, Ws2_ref,
            Wia1_ref, Wqa1_ref, va1_ref, W1c_ref,
            Wia2_ref, Wqa2_ref, va2_ref, w2c_ref,
            ctx1p_ref, ctx2s_ref):
    Wx = Wx_ref[:]
    Wh = Wh_ref[:]
    bb = b_ref[:]
    qlen = qlen_ref[:]

    def step(t, carry):
        h, c, hl = carry
        x = emb_ref[pl.ds(t * 8, 8), :]
        g = _mm(x, Wx) + _mm(h, Wh) + bb
        ii = jax.nn.sigmoid(g[:, 0:512])
        ff = jax.nn.sigmoid(g[:, 512:1024])
        gg = jnp.tanh(g[:, 1024:1536])
        oo = jax.nn.sigmoid(g[:, 1536:2048])
        c = ff * c + ii * gg
        h = oo * jnp.tanh(c)
        hl = jnp.where(qlen - 1 == t, h, hl)
        return (h, c, hl)

    z = jnp.zeros((8, 512), F32)
    _, _, que = lax.fori_loop(0, 20, step, (z, z, z))

    ST = _iota_eq((288, 8), lambda r: r // 36, lambda c: c)
    S = _iota_eq((8, 288), lambda r: r, lambda c: c // 36)
    STQ = _iota_eq((10368, 8), lambda r: r // 1296, lambda c: c)

    imgs_raw = img_ref[:]
    nlog = jnp.tanh(_mm(imgs_raw, Win_ref[:]) + _mm(ST, _mm(que, Wqn_ref[:])))
    en = jnp.exp(_mm(nlog, vn_ref[:]))
    nav = en / _mm(ST, _mm(S, en))
    imgs = nav * imgs_raw

    rel = rel_ref[:]
    t2 = jnp.tanh(_mm(rel, Wrr_ref[:]) + _mm(STQ, _mm(que, Wqr_ref[:])))
    rav = _mm(t2, vr_ref[:])
    rels = rav * rel

    S36 = _iota_eq((36, 1296), lambda r: r, lambda c: c // 36)
    OH = _iota_eq((1296, 36), lambda r: r % 36, lambda c: c)
    ravo = rav * jnp.concatenate([OH] * 8, axis=0)
    relagg = jnp.concatenate(
        [_mm(S36, rels[b * 1296:(b + 1) * 1296, :]) for b in range(8)], axis=0)
    ATT = jnp.concatenate(
        [_mm(S36, ravo[b * 1296:(b + 1) * 1296, :]) for b in range(8)], axis=0)
    G36 = _iota_eq((36, 288), lambda r: r, lambda c: c % 36)
    BD = _iota_eq((288, 288), lambda r: r // 36, lambda c: c // 36)
    P = _mm(ATT, G36) * BD

    agg1 = _mm(_mm(P, imgs), Wm1t_ref[:]) + _mm(relagg, Wm1b_ref[:])
    h_img = jax.nn.relu(_mm(imgs, Ws1_ref[:]) + agg1)
    agg2 = _mm(_mm(P, h_img), Wm2t_ref[:]) + _mm(relagg, Wm2b_ref[:])
    h_img2 = jax.nn.relu(_mm(h_img, Ws2_ref[:]) + agg2)

    ea = jnp.tanh(_mm(h_img, Wia1_ref[:]) + _mm(ST, _mm(que, Wqa1_ref[:])))
    e1 = jnp.exp(_mm(ea, va1_ref[:]))
    al1 = e1 / _mm(ST, _mm(S, e1))
    ctx1 = _mm(S, al1 * h_img)
    ctx1p_ref[:] = _mm(ctx1, W1c_ref[:])

    eb = jnp.tanh(_mm(h_img2, Wia2_ref[:]) + _mm(ST, _mm(que, Wqa2_ref[:])))
    e2 = jnp.exp(_mm(eb, va2_ref[:]))
    al2 = e2 / _mm(ST, _mm(S, e2))
    ctx2 = _mm(S, al2 * h_img2)
    ctx2s_ref[:] = _mm(ctx2, w2c_ref[:])


def _run_a(emb2, qlen, images2, rel2, *weights):
    return pl.pallas_call(
        _a_body,
        out_shape=(jax.ShapeDtypeStruct((8, 300), F32),
                   jax.ShapeDtypeStruct((8, 1), F32)),
    )(emb2, qlen, images2, rel2, *weights)


def _make_seg_sum(ffw, nwin, split_by_core, name, dtype=F32):
    ept = 25600 if split_by_core else 51200
    rows_pt = NP // NS
    out_rows = 2 * NP if split_by_core else 2 * nwin * NP
    mesh = plsc.VectorSubcoreMesh(core_axis_name="c", subcore_axis_name="s",
                                  num_cores=NC, num_subcores=NS)

    G = ECH // 128
    lanes = 32 if dtype == jnp.bfloat16 else 16

    @functools.partial(
        pl.kernel,
        out_type=jax.ShapeDtypeStruct((out_rows, ffw), dtype),
        mesh=mesh,
        scratch_types=(
            [pltpu.VMEM((ECH,), I32),
             pltpu.VMEM((ECH,), I32)]
            + [pltpu.VMEM((128,), I32) for _ in range(NBUF)]
            + [pltpu.VMEM((128,), I32) for _ in range(NBUF)]
            + [pltpu.VMEM((128, ffw), dtype) for _ in range(NBUF)]
            + [pltpu.VMEM((16, ffw), dtype)]
            + [pltpu.VMEM_SHARED((NP + 8, ffw), dtype)]
            + [pltpu.SemaphoreType.DMA for _ in range(2 * NBUF)]
        ),
        name=name,
        compiler_params=pltpu.CompilerParams(use_tc_tiling_on_sc=False),
    )
    def seg(e1_hbm, e2_hbm, tab_hbm, out_hbm, te1, te2, *sc):
        ce1s = sc[0:NBUF]
        ce2s = sc[NBUF:2 * NBUF]
        rowss = sc[2 * NBUF:3 * NBUF]
        zb = sc[3 * NBUF]
        acc = sc[3 * NBUF + 1]
        gsems = sc[3 * NBUF + 2:3 * NBUF + 2 + NBUF]
        ssems = sc[3 * NBUF + 2 + NBUF:3 * NBUF + 2 + 2 * NBUF]
        c = lax.axis_index("c")
        s = lax.axis_index("s")
        zerov = jnp.zeros((lanes,), dtype)
        dumpv = jnp.full((16,), NP, I32)
        for r in range(16):
            for cc in range(ffw // lanes):
                zb[r, pl.ds(cc * lanes, lanes)] = zerov

        for w in range(nwin):
            g = c * nwin + w
            goff = 0 if split_by_core else g * NP

            def zbody(z, _):
                off = pl.multiple_of(s * rows_pt + z * 16, 16)
                pltpu.sync_copy(zb, acc.at[pl.ds(off, 16)])
                return 0
            lax.fori_loop(0, rows_pt // 16, zbody, 0)
            plsc.subcore_barrier()

            if split_by_core:
                tbase = (s * NC + c) * ept
            else:
                tbase = s * ept
            nch = jnp.minimum(ept, jnp.maximum(E_FACT - tbase, 0)) // ECH

            def chunk(ch, _):
                base = pl.multiple_of(tbase + ch * ECH, 8)
                pltpu.sync_copy(e1_hbm.at[pl.ds(base, ECH)], te1)
                pltpu.sync_copy(e2_hbm.at[pl.ds(base, ECH)], te2)

                def scat(k):
                    b = k % NBUF
                    return pltpu.async_copy(rowss[b], acc.at[ce2s[b]],
                                            ssems[b], add=True)

                gds, sds = {}, {}
                for k in range(G):
                    b = k % NBUF
                    if k >= NBUF:
                        sds[k - NBUF].wait()
                    for v in range(8):
                        o = k * 128 + v * 16
                        ev1 = te1[pl.ds(o, 16)]
                        ev2 = te2[pl.ds(o, 16)]
                        ce1s[b][pl.ds(v * 16, 16)] = ev1 + goff
                        ce2s[b][pl.ds(v * 16, 16)] = jnp.where(ev2 < NP,
                                                               ev2, dumpv)
                    gds[k] = pltpu.async_copy(tab_hbm.at[ce1s[b]], rowss[b],
                                              gsems[b])
                    kp = k - (NBUF - 1)
                    if kp >= 0:
                        gds[kp].wait()
                        sds[kp] = scat(kp)
                for kp in range(max(0, G - (NBUF - 1)), G):
                    gds[kp].wait()
                    sds[kp] = scat(kp)
                for kp in range(max(0, G - NBUF), G):
                    sds[kp].wait()
                return 0
            lax.fori_loop(0, nch, chunk, 0)

            plsc.subcore_barrier()
            src = pl.multiple_of(s * rows_pt, 8)
            if split_by_core:
                dst = c * NP + s * rows_pt
            else:
                dst = g * NP + s * rows_pt
            dst = pl.multiple_of(dst, 8)
            pltpu.sync_copy(acc.at[pl.ds(src, rows_pt)],
                            out_hbm.at[pl.ds(dst, rows_pt)])
            plsc.subcore_barrier()

    return seg


@functools.cache
def _seg_wide():
    return _make_seg_sum(GWB, 1, False, "seg_sum_gcn1", jnp.bfloat16)


@functools.cache
def _seg_scalar():
    return _make_seg_sum(SW, 1, True, "seg_sum_gcn2")


def _seg_sum_sc_wide(e1p, e2p, table):
    return _seg_wide()(e1p, e2p, table)


def _seg_sum_sc_scalar(e1p, e2p, table):
    out = _seg_scalar()(e1p, e2p, table)
    return out[:NP, 1:2], out[NP:, 1:2]


def _c_body(f0_ref, f1_ref, a0_ref, a1_ref,
            fb_ref, W1h_ref, W1a_ref, c1p_ref, wv2_ref, c2s_ref, out_ref):
    oh = (fb_ref[:] == lax.broadcasted_iota(I32, (512, 8), 1)).astype(F32)
    ff = jnp.concatenate([f0_ref[:], f1_ref[:]], axis=1).astype(F32)
    agg = jnp.concatenate([a0_ref[:], a1_ref[:]], axis=1).astype(F32)
    h1 = jax.nn.relu(_mm(ff, W1h_ref[:]) + _mm(agg, W1a_ref[:])
                     + _mm(oh, c1p_ref[:]))
    hw = _mm(h1, wv2_ref[:])
    out_ref[:, 0:1] = hw[:, 0:1] + _mm(oh, c2s_ref[:])
    out_ref[:, 1:2] = hw[:, 1:2]
    out_ref[:, 2:SW] = jnp.zeros((512, SW - 2), F32)


def _run_c(tab16, agg_stacked, fb2, W1hp, W1ap, ctx1p, wv2, ctx2s):
    full = lambda shape: pl.BlockSpec(shape, lambda i: (0, 0))
    row = lambda shape: pl.BlockSpec(shape, lambda i: (i, 0))
    gblk = lambda g: pl.BlockSpec((512, GWB), lambda i, g=g: (g * (NP // 512) + i, 0))
    return pl.pallas_call(
        _c_body,
        grid=(NP // 512,),
        in_specs=[gblk(0), gblk(1), gblk(0), gblk(1),
                  row((512, 1)),
                  full((FFW, 300)), full((FFW, 300)), full((8, 300)),
                  full((300, 2)), full((8, 1))],
        out_specs=row((512, SW)),
        out_shape=jax.ShapeDtypeStruct((NP, SW), F32),
    )(tab16, tab16, agg_stacked, agg_stacked,
      fb2, W1hp, W1ap, ctx1p, wv2, ctx2s)


def _e_body(t_ref, pa_ref, pb_ref, o_ref):
    x = jax.nn.sigmoid(t_ref[:] + pa_ref[:] + pb_ref[:])
    gid = (lax.broadcasted_iota(I32, (NP // 128, 128), 0) * 128
           + lax.broadcasted_iota(I32, (NP // 128, 128), 1))
    e = jnp.where(gid < N_FACT, jnp.exp(x), 0.0)
    o_ref[:] = e / jnp.sum(e)


def _run_e(t2, pa2, pb2):
    return pl.pallas_call(
        _e_body,
        out_shape=jax.ShapeDtypeStruct((NP // 128, 128), F32),
    )(t2, pa2, pb2)


def kernel(questions, question_length, images, img_relations, fact_features,
           fact_e1ids, fact_e2ids, fact_batch, glove, W_x, W_h, b_lstm,
           Wq_node, Wi_node, v_node, Wq_rel, Wr_rel, v_rel, W_msg1, W_self1,
           W_msg2, W_self2, W_img_att1, W_que_att1, v_att1, W_fact1,
           W_img_att2, W_que_att2, v_att2, W_fact2):
    emb = jnp.take(glove, questions, axis=0)
    emb2 = jnp.swapaxes(emb, 0, 1).reshape(160, 300)
    qlen = question_length.astype(I32).reshape(8, 1)
    images2 = images.reshape(288, 2048)
    rel2 = jnp.swapaxes(img_relations, 1, 2).reshape(10368, 7)

    weights = (
        W_x, W_h, b_lstm.reshape(1, 2048), Wq_node, Wi_node, v_node,
        Wq_rel, Wr_rel, v_rel,
        W_msg1[:2048], W_msg1[2048:], W_self1,
        W_msg2[:512], W_msg2[512:], W_self2,
        W_img_att1, W_que_att1, v_att1, W_fact1[200:],
        W_img_att2, W_que_att2, v_att2, W_fact2[600:],
    )
    ctx1p, ctx2s = _run_a(emb2, qlen, images2, rel2, *weights)

    e1p = fact_e1ids.astype(I32)
    e2p = fact_e2ids.astype(I32)
    tab = jnp.concatenate(
        [jnp.pad(fact_features[:, g * GWB:min((g + 1) * GWB, 100)],
                 ((0, NP - N_FACT), (0, max(0, (g + 1) * GWB - 100))))
         for g in range(2)]).astype(jnp.bfloat16)

    agg1 = _seg_sum_sc_wide(e1p, e2p, tab)

    fb2 = jnp.pad(fact_batch.astype(I32), (0, NP - N_FACT)).reshape(NP, 1)
    W1hp = jnp.pad(W_fact1[:100], ((0, FFW - 100), (0, 0)))
    W1ap = jnp.pad(W_fact1[100:200], ((0, FFW - 100), (0, 0)))
    wv2 = jnp.concatenate([W_fact2[:300], W_fact2[300:600]], axis=1)
    ts = _run_c(tab, agg1, fb2, W1hp, W1ap, ctx1p, wv2, ctx2s)

    pa, pb = _seg_sum_sc_scalar(e1p, e2p, ts)

    out = _run_e(ts[:, 0:1].reshape(NP // 128, 128),
                 pa.reshape(NP // 128, 128), pb.reshape(NP // 128, 128))
    return out.reshape(NP, 1)[:N_FACT]
